# baseline (speedup 1.0000x reference)
import functools

import jax
import jax.numpy as jnp
from jax.experimental import pallas as pl
from jax.experimental.pallas import tpu as pltpu


def _dot_f32(a, b):
    return jax.lax.dot_general(
        a, b, (((1,), (0,)), ((), ())),
        precision=jax.lax.Precision.DEFAULT,
        preferred_element_type=jnp.float32)


def _wt_xt(w, x):
    return jax.lax.dot_general(
        w, jnp.maximum(x, 0.0), (((0,), (1,)), ((), ())),
        precision=jax.lax.Precision.DEFAULT,
        preferred_element_type=jnp.float32)


def _nt_dot(at, n):
    return jax.lax.dot_general(
        at, n, (((1,), (1,)), ((), ())),
        precision=jax.lax.Precision.DEFAULT,
        preferred_element_type=jnp.float32)


def _node_kernel(n_ref, x0_ref, w0_ref, w1_ref, o_ref,
                 a0t_ref, a1t_ref, x0at_ref, *, bm):
    p = pl.program_id(0)
    i = pl.program_id(1)

    @pl.when((p == 0) & (i == 0))
    def _():
        a0t_ref[:] = _wt_xt(w0_ref[:], x0_ref[:])

    @pl.when(p == 0)
    def _():
        blk = jnp.maximum(_nt_dot(a0t_ref[:], n_ref[:]), 0.0)
        x0at_ref[:, pl.ds(i * bm, bm)] = blk
        o_ref[:] = jnp.sum(blk, axis=1, keepdims=True)[None]

    @pl.when((p == 1) & (i == 0))
    def _():
        a1t_ref[:] = jax.lax.dot_general(
            w1_ref[:], x0at_ref[:], (((0,), (0,)), ((), ())),
            precision=jax.lax.Precision.DEFAULT,
            preferred_element_type=jnp.float32)

    @pl.when(p == 1)
    def _():
        x0bt = jnp.maximum(_nt_dot(a1t_ref[:], n_ref[:]), 0.0)
        o_ref[:] = jnp.sum(x0bt, axis=1, keepdims=True)[None]


def _stream3_kernel(n_ref, x1_ref, w_ref, o_ref, s1_ref, bt_ref):
    @pl.when(pl.program_id(0) == 0)
    def _():
        bt_ref[:] = _wt_xt(w_ref[:], x1_ref[:])
        s1_ref[:] = jnp.sum(jnp.maximum(x1_ref[:], 0.0), axis=0,
                            keepdims=True)
    x2t = jnp.maximum(_nt_dot(bt_ref[:], n_ref[:]), 0.0)
    o_ref[:] = jnp.sum(x2t, axis=1, keepdims=True)[None]


def _final_kernel(s0_ref, s2_ref, s1_ref,
                  w0_ref, b0_ref, w1_ref, b1_ref, w2_ref, b2_ref, o_ref,
                  *, n_nodes, n_edges, n_faces, g1):
    m0 = jnp.sum(s0_ref[g1:], axis=0) / n_nodes
    m1 = s1_ref[:] / n_edges
    m2 = jnp.sum(s2_ref[:], axis=0) / n_faces
    o_ref[:] = (
        jax.lax.dot_general(m0, w0_ref[:], (((0,), (0,)), ((), ())),
                            preferred_element_type=jnp.float32)
        + b0_ref[:]
        + _dot_f32(m1, w1_ref[:]) + b1_ref[:]
        + jax.lax.dot_general(m2, w2_ref[:], (((0,), (0,)), ((), ())),
                              preferred_element_type=jnp.float32)
        + b2_ref[:])


_N_BUF = 2
_BM = 256


def _n_spec(bm, k):
    return pl.BlockSpec((bm, k), lambda i: (i, 0),
                        pipeline_mode=pl.Buffered(buffer_count=_N_BUF))


def kernel(x_0, x_1, neighborhood_0_to_0, neighborhood_1_to_2,
           w00_l0, w12_l0, w00_l1, w12_l1,
           lin0_w, lin0_b, lin1_w, lin1_b, lin2_w, lin2_b):
    n_nodes, c0 = x_0.shape
    n_edges, c1 = x_1.shape
    n_faces = neighborhood_1_to_2.shape[0]
    c2 = w12_l1.shape[1]
    ncls = lin0_w.shape[1]
    bm = _BM
    bm3 = 128
    g1 = n_nodes // bm
    g3 = n_faces // bm3

    params = pltpu.CompilerParams(dimension_semantics=("arbitrary", "arbitrary"))
    params1 = pltpu.CompilerParams(dimension_semantics=("arbitrary",))

    s0p = pl.pallas_call(
        functools.partial(_node_kernel, bm=bm),
        grid=(2, g1),
        in_specs=[
            pl.BlockSpec((bm, n_nodes), lambda p, i: (i, 0),
                         pipeline_mode=pl.Buffered(buffer_count=_N_BUF)),
            pl.BlockSpec((n_nodes, c0), lambda p, i: (0, 0)),
            pl.BlockSpec((c0, c0), lambda p, i: (0, 0)),
            pl.BlockSpec((c0, c0), lambda p, i: (0, 0)),
        ],
        out_specs=pl.BlockSpec((1, c0, 1), lambda p, i: (p * g1 + i, 0, 0)),
        out_shape=jax.ShapeDtypeStruct((2 * g1, c0, 1), jnp.float32),
        scratch_shapes=[pltpu.VMEM((c0, n_nodes), jnp.float32),
                        pltpu.VMEM((c0, n_nodes), jnp.float32),
                        pltpu.VMEM((c0, n_nodes), jnp.float32)],
        compiler_params=params,
    )(neighborhood_0_to_0, x_0, w00_l0, w00_l1)

    s2p, s1 = pl.pallas_call(
        _stream3_kernel,
        grid=(g3,),
        in_specs=[
            _n_spec(bm3, n_edges),
            pl.BlockSpec((n_edges, c1), lambda i: (0, 0)),
            pl.BlockSpec((c1, c2), lambda i: (0, 0)),
        ],
        out_specs=(
            pl.BlockSpec((1, c2, 1), lambda i: (i, 0, 0)),
            pl.BlockSpec((1, c1), lambda i: (0, 0)),
        ),
        out_shape=(
            jax.ShapeDtypeStruct((g3, c2, 1), jnp.float32),
            jax.ShapeDtypeStruct((1, c1), jnp.float32),
        ),
        scratch_shapes=[pltpu.VMEM((c2, n_edges), jnp.float32)],
        compiler_params=params1,
    )(neighborhood_1_to_2, x_1, w12_l1)

    final = functools.partial(_final_kernel, n_nodes=float(n_nodes),
                              n_edges=float(n_edges), n_faces=float(n_faces),
                              g1=g1)
    out = pl.pallas_call(
        final,
        out_shape=jax.ShapeDtypeStruct((1, ncls), jnp.float32),
    )(s0p, s2p, s1,
      lin0_w, lin0_b.reshape(1, ncls), lin1_w, lin1_b.reshape(1, ncls),
      lin2_w, lin2_b.reshape(1, ncls))
    return out.reshape(ncls)


# speedup vs baseline: 1.0089x; 1.0089x over previous
import functools

import jax
import jax.numpy as jnp
from jax.experimental import pallas as pl
from jax.experimental.pallas import tpu as pltpu


def _wt_xt(w, x):
    return jax.lax.dot_general(
        w, jnp.maximum(x, 0.0), (((0,), (1,)), ((), ())),
        precision=jax.lax.Precision.DEFAULT,
        preferred_element_type=jnp.float32)


def _nt_dot(at, n):
    return jax.lax.dot_general(
        at, n, (((1,), (1,)), ((), ())),
        precision=jax.lax.Precision.DEFAULT,
        preferred_element_type=jnp.float32)


def _node_kernel(n_ref, x0_ref, w0_ref, w1_ref, s0_ref,
                 a0t_ref, a1t_ref, x0at_ref, acc_ref, *, bm):
    p = pl.program_id(0)
    i = pl.program_id(1)

    @pl.when((p == 0) & (i == 0))
    def _():
        a0t_ref[:] = _wt_xt(w0_ref[:], x0_ref[:])

    @pl.when(p == 0)
    def _():
        x0at_ref[:, pl.ds(i * bm, bm)] = jnp.maximum(
            _nt_dot(a0t_ref[:], n_ref[:]), 0.0)

    @pl.when((p == 1) & (i == 0))
    def _():
        a1t_ref[:] = jax.lax.dot_general(
            w1_ref[:], x0at_ref[:], (((0,), (0,)), ((), ())),
            precision=jax.lax.Precision.DEFAULT,
            preferred_element_type=jnp.float32)
        acc_ref[:] = jnp.zeros_like(acc_ref)

    @pl.when(p == 1)
    def _():
        x0bt = jnp.maximum(_nt_dot(a1t_ref[:], n_ref[:]), 0.0)
        acc_ref[:] += jnp.sum(x0bt, axis=1, keepdims=True)
        s0_ref[:] = acc_ref[:]


def _face_kernel(n_ref, x1_ref, w12_ref, s0_ref,
                 w0_ref, b0_ref, w1_ref, b1_ref, w2_ref, b2_ref,
                 o_ref, bt_ref, s1_ref, acc_ref,
                 *, g3, n_nodes, n_edges, n_faces):
    i = pl.program_id(0)

    @pl.when(i == 0)
    def _():
        bt_ref[:] = _wt_xt(w12_ref[:], x1_ref[:])
        s1_ref[:] = jnp.sum(jnp.maximum(x1_ref[:], 0.0), axis=0,
                            keepdims=True)
        acc_ref[:] = jnp.zeros_like(acc_ref)

    x2t = jnp.maximum(_nt_dot(bt_ref[:], n_ref[:]), 0.0)
    acc_ref[:] += jnp.sum(x2t, axis=1, keepdims=True)

    @pl.when(i == g3 - 1)
    def _():
        m0 = s0_ref[:] / n_nodes
        m1 = s1_ref[:] / n_edges
        m2 = acc_ref[:] / n_faces
        o_ref[:] = (
            jax.lax.dot_general(m0, w0_ref[:], (((0,), (0,)), ((), ())),
                                preferred_element_type=jnp.float32)
            + b0_ref[:]
            + jax.lax.dot_general(m1, w1_ref[:], (((1,), (0,)), ((), ())),
                                  preferred_element_type=jnp.float32)
            + b1_ref[:]
            + jax.lax.dot_general(m2, w2_ref[:], (((0,), (0,)), ((), ())),
                                  preferred_element_type=jnp.float32)
            + b2_ref[:])


def kernel(x_0, x_1, neighborhood_0_to_0, neighborhood_1_to_2,
           w00_l0, w12_l0, w00_l1, w12_l1,
           lin0_w, lin0_b, lin1_w, lin1_b, lin2_w, lin2_b):
    n_nodes, c0 = x_0.shape
    n_edges, c1 = x_1.shape
    n_faces = neighborhood_1_to_2.shape[0]
    c2 = w12_l1.shape[1]
    ncls = lin0_w.shape[1]
    bm = 256
    bm3 = 128
    g1 = n_nodes // bm
    g3 = n_faces // bm3

    s0 = pl.pallas_call(
        functools.partial(_node_kernel, bm=bm),
        grid=(2, g1),
        in_specs=[
            pl.BlockSpec((bm, n_nodes), lambda p, i: (i, 0)),
            pl.BlockSpec((n_nodes, c0), lambda p, i: (0, 0)),
            pl.BlockSpec((c0, c0), lambda p, i: (0, 0)),
            pl.BlockSpec((c0, c0), lambda p, i: (0, 0)),
        ],
        out_specs=pl.BlockSpec((c0, 1), lambda p, i: (0, 0)),
        out_shape=jax.ShapeDtypeStruct((c0, 1), jnp.float32),
        scratch_shapes=[pltpu.VMEM((c0, n_nodes), jnp.float32),
                        pltpu.VMEM((c0, n_nodes), jnp.float32),
                        pltpu.VMEM((c0, n_nodes), jnp.float32),
                        pltpu.VMEM((c0, 1), jnp.float32)],
        compiler_params=pltpu.CompilerParams(
            dimension_semantics=("arbitrary", "arbitrary")),
    )(neighborhood_0_to_0, x_0, w00_l0, w00_l1)

    face = functools.partial(_face_kernel, g3=g3, n_nodes=float(n_nodes),
                             n_edges=float(n_edges), n_faces=float(n_faces))
    out = pl.pallas_call(
        face,
        grid=(g3,),
        in_specs=[
            pl.BlockSpec((bm3, n_edges), lambda i: (i, 0)),
            pl.BlockSpec((n_edges, c1), lambda i: (0, 0)),
            pl.BlockSpec((c1, c2), lambda i: (0, 0)),
            pl.BlockSpec((c0, 1), lambda i: (0, 0)),
            pl.BlockSpec((c0, ncls), lambda i: (0, 0)),
            pl.BlockSpec((1, ncls), lambda i: (0, 0)),
            pl.BlockSpec((c1, ncls), lambda i: (0, 0)),
            pl.BlockSpec((1, ncls), lambda i: (0, 0)),
            pl.BlockSpec((c2, ncls), lambda i: (0, 0)),
            pl.BlockSpec((1, ncls), lambda i: (0, 0)),
        ],
        out_specs=pl.BlockSpec((1, ncls), lambda i: (0, 0)),
        out_shape=jax.ShapeDtypeStruct((1, ncls), jnp.float32),
        scratch_shapes=[pltpu.VMEM((c2, n_edges), jnp.float32),
                        pltpu.VMEM((1, c1), jnp.float32),
                        pltpu.VMEM((c2, 1), jnp.float32)],
        compiler_params=pltpu.CompilerParams(
            dimension_semantics=("arbitrary",)),
    )(neighborhood_1_to_2, x_1, w12_l1, s0,
      lin0_w, lin0_b.reshape(1, ncls), lin1_w, lin1_b.reshape(1, ncls),
      lin2_w, lin2_b.reshape(1, ncls))
    return out.reshape(ncls)
